# Initial kernel scaffold; baseline (speedup 1.0000x reference)
import jax
import jax.numpy as jnp
from jax import lax
from jax.experimental import pallas as pl
from jax.experimental.pallas import tpu as pltpu
from jax.experimental.pallas import tpu_sc as plsc

B = 16384
EMBED_DIM = 32
NUM_BUCKETS = 1000
PAD_BUCKETS = 1024
NC, NS, L = 2, 16, 16
NW = NC * NS
CHUNK = B // NW
GATHER_W = 128
N_GATHERS = CHUNK // GATHER_W


def _sc_body(uidx_hbm, ts_hbm, utab_hbm, ttab_hbm, bnd_hbm, mean_hbm, std_hbm,
             out_hbm,
             uidx_v, ts_v, bnd_v, bidx_v, urows_v, trows_v, norm_v,
             mean_v, std_v, gsem, osem):
    wid = lax.axis_index("s") * NC + lax.axis_index("c")
    base = wid * CHUNK

    pltpu.sync_copy(uidx_hbm.at[pl.ds(base, CHUNK)], uidx_v)
    user_copies = []
    for j in range(N_GATHERS):
        user_copies.append(pltpu.async_copy(
            utab_hbm.at[uidx_v.at[pl.ds(j * GATHER_W, GATHER_W)]],
            urows_v.at[pl.ds(j * GATHER_W, GATHER_W)], gsem))

    pltpu.sync_copy(ts_hbm.at[pl.ds(base, CHUNK)], ts_v)
    pltpu.sync_copy(bnd_hbm, bnd_v.at[pl.ds(0, NUM_BUCKETS)])
    pltpu.sync_copy(mean_hbm, mean_v)
    pltpu.sync_copy(std_hbm, std_v)
    inf16 = jnp.full((L,), jnp.inf, jnp.float32)
    bnd_v[pl.ds(NUM_BUCKETS, L)] = inf16
    bnd_v[pl.ds(PAD_BUCKETS - L, L)] = inf16
    mean16 = mean_v[...]
    std16 = std_v[...]

    for i in range(CHUNK // L):
        ts = ts_v[pl.ds(i * L, L)]
        pos = jnp.zeros((L,), jnp.int32)
        step = PAD_BUCKETS // 2
        while step >= 1:
            cand = pos + step
            val = plsc.load_gather(bnd_v, [cand - 1])
            pos = jnp.where(val <= ts, cand, pos)
            step //= 2
        bidx_v[pl.ds(i * L, L)] = pos
        nrm = (ts - mean16) / std16
        rows = lax.iota(jnp.int32, L) + (i * L)
        plsc.store_scatter(norm_v, [rows, jnp.zeros((L,), jnp.int32)], nrm)

    ts_copies = []
    for j in range(N_GATHERS):
        ts_copies.append(pltpu.async_copy(
            ttab_hbm.at[bidx_v.at[pl.ds(j * GATHER_W, GATHER_W)]],
            trows_v.at[pl.ds(j * GATHER_W, GATHER_W)], gsem))
    for c in user_copies + ts_copies:
        c.wait()

    o1 = pltpu.async_copy(
        urows_v, out_hbm.at[pl.ds(base, CHUNK), pl.ds(0, EMBED_DIM)], osem)
    o2 = pltpu.async_copy(
        trows_v, out_hbm.at[pl.ds(base, CHUNK), pl.ds(EMBED_DIM, EMBED_DIM)],
        osem)
    o3 = pltpu.async_copy(
        norm_v, out_hbm.at[pl.ds(base, CHUNK), pl.ds(2 * EMBED_DIM, 1)], osem)
    o1.wait()
    o2.wait()
    o3.wait()


def kernel(user_idx, timestamp, user_table, ts_table, boundaries, ts_mean,
           ts_var):
    mesh = plsc.VectorSubcoreMesh(core_axis_name="c", subcore_axis_name="s")
    std16 = jnp.broadcast_to(jnp.sqrt(ts_var), (L,)).astype(jnp.float32)
    mean16 = jnp.broadcast_to(ts_mean, (L,)).astype(jnp.float32)
    sc = pl.kernel(
        _sc_body,
        out_type=jax.ShapeDtypeStruct((B, 2 * EMBED_DIM + 1), jnp.float32),
        mesh=mesh,
        scratch_types=[
            pltpu.VMEM((CHUNK,), jnp.int32),
            pltpu.VMEM((CHUNK,), jnp.float32),
            pltpu.VMEM((PAD_BUCKETS,), jnp.float32),
            pltpu.VMEM((CHUNK,), jnp.int32),
            pltpu.VMEM((CHUNK, EMBED_DIM), jnp.float32),
            pltpu.VMEM((CHUNK, EMBED_DIM), jnp.float32),
            pltpu.VMEM((CHUNK, 1), jnp.float32),
            pltpu.VMEM((L,), jnp.float32),
            pltpu.VMEM((L,), jnp.float32),
            pltpu.SemaphoreType.DMA,
            pltpu.SemaphoreType.DMA,
        ],
    )
    return sc(user_idx.astype(jnp.int32), timestamp.astype(jnp.float32),
              user_table, ts_table, boundaries, mean16, std16)


# speedup vs baseline: 2.0058x; 2.0058x over previous
import jax
import jax.numpy as jnp
from jax import lax
from jax.experimental import pallas as pl
from jax.experimental.pallas import tpu as pltpu
from jax.experimental.pallas import tpu_sc as plsc

B = 16384
EMBED_DIM = 32
NUM_BUCKETS = 1000
PAD_BUCKETS = 1024
NC, NS, L = 2, 16, 16
NW = NC * NS
CHUNK = B // NW
GATHER_W = 128
N_GATHERS = CHUNK // GATHER_W


def _sc_body(uidx_hbm, ts_hbm, utab_hbm, ttab_hbm, bnd_hbm, mean_hbm, std_hbm,
             out_hbm,
             uidx_v, ts_v, bnd_v, bidx_v, urows_v, trows_v, norm_v,
             mean_v, std_v, gsem, osem):
    wid = lax.axis_index("s") * NC + lax.axis_index("c")
    base = wid * CHUNK

    pltpu.sync_copy(uidx_hbm.at[pl.ds(base, CHUNK)], uidx_v)
    user_copies = []
    for j in range(N_GATHERS):
        user_copies.append(pltpu.async_copy(
            utab_hbm.at[uidx_v.at[pl.ds(j * GATHER_W, GATHER_W)]],
            urows_v.at[pl.ds(j * GATHER_W, GATHER_W)], gsem))

    pltpu.sync_copy(ts_hbm.at[pl.ds(base, CHUNK)], ts_v)
    pltpu.sync_copy(bnd_hbm, bnd_v.at[pl.ds(0, NUM_BUCKETS)])
    pltpu.sync_copy(mean_hbm, mean_v)
    pltpu.sync_copy(std_hbm, std_v)
    inf16 = jnp.full((L,), jnp.inf, jnp.float32)
    bnd_v[pl.ds(NUM_BUCKETS, L)] = inf16
    bnd_v[pl.ds(PAD_BUCKETS - L, L)] = inf16
    mean16 = mean_v[...]
    std16 = std_v[...]

    for i in range(CHUNK // L):
        ts = ts_v[pl.ds(i * L, L)]
        pos = jnp.zeros((L,), jnp.int32)
        step = PAD_BUCKETS // 2
        while step >= 1:
            cand = pos + step
            val = plsc.load_gather(bnd_v, [cand - 1])
            pos = jnp.where(val <= ts, cand, pos)
            step //= 2
        bidx_v[pl.ds(i * L, L)] = pos
        nrm = (ts - mean16) / std16
        rows = lax.iota(jnp.int32, L) + (i * L)
        plsc.store_scatter(norm_v, [rows, jnp.zeros((L,), jnp.int32)], nrm)

    ts_copies = []
    for j in range(N_GATHERS):
        ts_copies.append(pltpu.async_copy(
            ttab_hbm.at[bidx_v.at[pl.ds(j * GATHER_W, GATHER_W)]],
            trows_v.at[pl.ds(j * GATHER_W, GATHER_W)], gsem))
    for c in user_copies + ts_copies:
        c.wait()

    o1 = pltpu.async_copy(
        urows_v, out_hbm.at[pl.ds(base, CHUNK), pl.ds(0, EMBED_DIM)], osem)
    o2 = pltpu.async_copy(
        trows_v, out_hbm.at[pl.ds(base, CHUNK), pl.ds(EMBED_DIM, EMBED_DIM)],
        osem)
    o3 = pltpu.async_copy(
        norm_v, out_hbm.at[pl.ds(base, CHUNK), pl.ds(2 * EMBED_DIM, 1)], osem)
    o1.wait()
    o2.wait()
    o3.wait()


def kernel(user_idx, timestamp, user_table, ts_table, boundaries, ts_mean,
           ts_var):
    mesh = plsc.VectorSubcoreMesh(core_axis_name="c", subcore_axis_name="s")
    std16 = jnp.broadcast_to(jnp.sqrt(ts_var), (L,)).astype(jnp.float32)
    mean16 = jnp.broadcast_to(ts_mean, (L,)).astype(jnp.float32)
    sc = pl.kernel(
        _sc_body,
        out_type=jax.ShapeDtypeStruct((B, 2 * EMBED_DIM + 1), jnp.float32),
        mesh=mesh,
        compiler_params=pltpu.CompilerParams(use_tc_tiling_on_sc=False,
                                             needs_layout_passes=False),
        scratch_types=[
            pltpu.VMEM((CHUNK,), jnp.int32),
            pltpu.VMEM((CHUNK,), jnp.float32),
            pltpu.VMEM((PAD_BUCKETS,), jnp.float32),
            pltpu.VMEM((CHUNK,), jnp.int32),
            pltpu.VMEM((CHUNK, EMBED_DIM), jnp.float32),
            pltpu.VMEM((CHUNK, EMBED_DIM), jnp.float32),
            pltpu.VMEM((CHUNK, 1), jnp.float32),
            pltpu.VMEM((L,), jnp.float32),
            pltpu.VMEM((L,), jnp.float32),
            pltpu.SemaphoreType.DMA,
            pltpu.SemaphoreType.DMA,
        ],
    )
    return sc(user_idx.astype(jnp.int32), timestamp.astype(jnp.float32),
              user_table, ts_table, boundaries, mean16, std16)
